# Initial kernel scaffold; baseline (speedup 1.0000x reference)
import functools

import jax
import jax.numpy as jnp
from jax import lax
from jax.experimental import pallas as pl
from jax.experimental.pallas import tpu as pltpu
from jax.experimental.pallas import tpu_sc as plsc

N = 50000
E = 800000
L = 8
TAG_V = 128
ATTR_V = 256
ED_TAG = 16
ED_ATTR = 32
H = 48
GC = 64
C = 16
B = 512

BLK = 2048
GRID = 25
NP = BLK * GRID
NSC = 2
NTILE = 16
CHUNK = 128
JROWS = 8
EP = 819200
ROWS_CONV = EP // NTILE // CHUNK
ROWS_HIST = EP // (NSC * NTILE) // CHUNK
STRIPE = NP // NTILE

_mesh = plsc.VectorSubcoreMesh(
    core_axis_name="c", subcore_axis_name="s", num_cores=NSC,
    num_subcores=NTILE)


def _sc_hist(col_idx, zeros16, ones16):
  @functools.partial(
      pl.kernel,
      out_type=jax.ShapeDtypeStruct((NSC, NP, 16), jnp.float32),
      mesh=_mesh,
      scratch_types=[
          pltpu.VMEM((JROWS, CHUNK), jnp.int32),
          pltpu.VMEM((CHUNK, 16), jnp.float32),
          pltpu.VMEM((CHUNK, 16), jnp.float32),
          pltpu.VMEM_SHARED((NP, 16), jnp.float32),
      ],
  )
  def k(col_hbm, z_hbm, ones_hbm, out_hbm, cv, zb, onesb, acc):
    cid = lax.axis_index("c")
    sid = lax.axis_index("s")
    pltpu.sync_copy(z_hbm, zb)
    pltpu.sync_copy(ones_hbm, onesb)

    @pl.loop(0, STRIPE // CHUNK)
    def _(i):
      pltpu.sync_copy(zb, acc.at[pl.ds(sid * STRIPE + i * CHUNK, CHUNK)])

    plsc.subcore_barrier()

    @pl.loop(0, ROWS_HIST // JROWS)
    def _(ci):
      pltpu.sync_copy(col_hbm.at[cid, sid, pl.ds(ci * JROWS, JROWS)], cv)
      for j in range(JROWS):
        pltpu.sync_copy(onesb, acc.at[cv.at[j]], add=True)

    plsc.subcore_barrier()
    pltpu.sync_copy(acc.at[pl.ds(sid * STRIPE, STRIPE)],
                    out_hbm.at[cid, pl.ds(sid * STRIPE, STRIPE)])

  return k(col_idx, zeros16, ones16)


def _sc_conv(y_lo, y_hi, row_idx, col_idx, zeros32):
  @functools.partial(
      pl.kernel,
      out_type=(jax.ShapeDtypeStruct((NP, 32), jnp.float32),
                jax.ShapeDtypeStruct((NP, 32), jnp.float32)),
      mesh=_mesh,
      scratch_types=[
          pltpu.VMEM((JROWS, CHUNK), jnp.int32),
          pltpu.VMEM((JROWS, CHUNK), jnp.int32),
          pltpu.VMEM((CHUNK, 32), jnp.float32),
          pltpu.VMEM((CHUNK, 32), jnp.float32),
          pltpu.VMEM_SHARED((NP, 32), jnp.float32),
      ],
  )
  def k(ylo_hbm, yhi_hbm, row_hbm, col_hbm, z_hbm, olo_hbm, ohi_hbm,
        rv, cv, gb, zb, acc):
    cid = lax.axis_index("c")
    sid = lax.axis_index("s")
    pltpu.sync_copy(z_hbm, zb)

    @pl.loop(0, STRIPE // CHUNK)
    def _(i):
      pltpu.sync_copy(zb, acc.at[pl.ds(sid * STRIPE + i * CHUNK, CHUNK)])

    plsc.subcore_barrier()

    def edge_pass(y_hbm):
      @pl.loop(0, ROWS_CONV // JROWS)
      def _(ci):
        pltpu.sync_copy(row_hbm.at[sid, pl.ds(ci * JROWS, JROWS)], rv)
        pltpu.sync_copy(col_hbm.at[sid, pl.ds(ci * JROWS, JROWS)], cv)
        for j in range(JROWS):
          pltpu.sync_copy(y_hbm.at[rv.at[j]], gb)
          pltpu.sync_copy(gb, acc.at[cv.at[j]], add=True)

    @pl.when(cid == 0)
    def _():
      edge_pass(ylo_hbm)

    @pl.when(cid == 1)
    def _():
      edge_pass(yhi_hbm)

    plsc.subcore_barrier()

    @pl.when(cid == 0)
    def _():
      pltpu.sync_copy(acc.at[pl.ds(sid * STRIPE, STRIPE)],
                      olo_hbm.at[pl.ds(sid * STRIPE, STRIPE)])

    @pl.when(cid == 1)
    def _():
      pltpu.sync_copy(acc.at[pl.ds(sid * STRIPE, STRIPE)],
                      ohi_hbm.at[pl.ds(sid * STRIPE, STRIPE)])

  return k(y_lo, y_hi, row_idx, col_idx, zeros32)


def _dinv_from_hist(hist_blk):
  deg = hist_blk[0, :, 0:1] + hist_blk[1, :, 0:1] + 1.0
  return lax.rsqrt(deg)


def _features_body(tags_ref, attrs_ref, tag_t_ref, attr_t_ref, wih_ref,
                   whh_ref, bias_ref, x_ref):
  tags = tags_ref[0, 0, :]
  oh_t = (tags[:, None] == lax.broadcasted_iota(jnp.int32, (BLK, TAG_V), 1)
          ).astype(jnp.float32)
  tag_e = jnp.dot(oh_t, tag_t_ref[...], preferred_element_type=jnp.float32)

  wih = wih_ref[...]
  whh = whh_ref[...]
  bias = bias_ref[...]
  h = jnp.zeros((BLK, H), jnp.float32)
  c = jnp.zeros((BLK, H), jnp.float32)
  for l in range(L):
    al = attrs_ref[0, l, :]
    oh_a = (al[:, None] == lax.broadcasted_iota(jnp.int32, (BLK, ATTR_V), 1)
            ).astype(jnp.float32)
    xt = jnp.dot(oh_a, attr_t_ref[...], preferred_element_type=jnp.float32)
    g = (lax.dot_general(xt, wih, (((1,), (1,)), ((), ())),
                         preferred_element_type=jnp.float32)
         + lax.dot_general(h, whh, (((1,), (1,)), ((), ())),
                           preferred_element_type=jnp.float32)
         + bias)
    gi = jax.nn.sigmoid(g[:, 0:H])
    gf = jax.nn.sigmoid(g[:, H:2 * H])
    gg = jnp.tanh(g[:, 2 * H:3 * H])
    go = jax.nn.sigmoid(g[:, 3 * H:4 * H])
    c = gf * c + gi * gg
    h = go * jnp.tanh(c)
  x_ref[...] = jnp.concatenate([tag_e, h], axis=1)


def _prep1_body(x_ref, hist_ref, w1_ref, ylo_ref, yhi_ref):
  dinv = _dinv_from_hist(hist_ref[...])
  xw = lax.dot_general(x_ref[...], w1_ref[...], (((1,), (1,)), ((), ())),
                       preferred_element_type=jnp.float32)
  y = dinv * xw
  ylo_ref[...] = y[:, :32]
  yhi_ref[...] = y[:, 32:]


def _mid_body(alo_ref, ahi_ref, ylo_ref, yhi_ref, hist_ref, b1_ref, w2_ref,
              y2lo_ref, y2hi_ref):
  dinv = _dinv_from_hist(hist_ref[...])
  acc = jnp.concatenate([alo_ref[...], ahi_ref[...]], axis=1)
  y1 = jnp.concatenate([ylo_ref[...], yhi_ref[...]], axis=1)
  h1 = jax.nn.relu(dinv * (acc + y1) + b1_ref[...])
  xw2 = lax.dot_general(h1, w2_ref[...], (((1,), (1,)), ((), ())),
                        preferred_element_type=jnp.float32)
  y2 = dinv * xw2
  y2lo_ref[...] = y2[:, :32]
  y2hi_ref[...] = y2[:, 32:]


def _pool_body(alo_ref, ahi_ref, ylo_ref, yhi_ref, hist_ref, b2_ref,
               batch_ref, wfc_ref, bfc_ref, out_ref, acc_ref):
  i = pl.program_id(0)

  @pl.when(i == 0)
  def _():
    acc_ref[...] = jnp.zeros_like(acc_ref)

  dinv = _dinv_from_hist(hist_ref[...])
  acc = jnp.concatenate([alo_ref[...], ahi_ref[...]], axis=1)
  y2 = jnp.concatenate([ylo_ref[...], yhi_ref[...]], axis=1)
  h2 = jax.nn.relu(dinv * (acc + y2) + b2_ref[...])
  h2a = jnp.concatenate([h2, jnp.ones((BLK, 8), jnp.float32)], axis=1)
  bb = batch_ref[0, 0, :]
  oh = (bb[:, None] == lax.broadcasted_iota(jnp.int32, (BLK, B), 1)
        ).astype(jnp.float32)
  acc_ref[...] += lax.dot_general(oh, h2a, (((0,), (0,)), ((), ())),
                                  preferred_element_type=jnp.float32)

  @pl.when(i == GRID - 1)
  def _():
    sums = acc_ref[..., :GC]
    cnt = jnp.maximum(acc_ref[..., GC:GC + 1], 1.0)
    pooled = sums / cnt
    out_ref[...] = (lax.dot_general(
        pooled, wfc_ref[...], (((1,), (1,)), ((), ())),
        preferred_element_type=jnp.float32) + bfc_ref[...])


def _full(shape):
  return pl.BlockSpec(shape, lambda i: tuple(0 for _ in shape))


def _rows(width):
  return pl.BlockSpec((BLK, width), lambda i: (i, 0))


def kernel(tags, attrs, edge_index, batch, tag_table, attr_table, W_ih, W_hh,
           b_ih, b_hh, W1, b1, W2, b2, Wfc, bfc):
  f32 = jnp.float32
  tags_p = jnp.pad(tags.astype(jnp.int32), (0, NP - N)).reshape(GRID, 1, BLK)
  attrs_p = jnp.pad(attrs.astype(jnp.int32), ((0, NP - N), (0, 0)))
  attrs_p = attrs_p.T.reshape(L, GRID, BLK).transpose(1, 0, 2)
  batch_p = jnp.pad(batch.astype(jnp.int32), (0, NP - N),
                    constant_values=B).reshape(GRID, 1, BLK)
  row = jnp.pad(edge_index[0].astype(jnp.int32), (0, EP - E))
  col = jnp.pad(edge_index[1].astype(jnp.int32), (0, EP - E),
                constant_values=NP - 1)
  row_c = row.reshape(NTILE, ROWS_CONV, CHUNK)
  col_c = col.reshape(NTILE, ROWS_CONV, CHUNK)
  col_h = col.reshape(NSC, NTILE, ROWS_HIST, CHUNK)
  ones16 = jnp.ones((CHUNK, 16), f32)
  zeros16 = jnp.zeros((CHUNK, 16), f32)
  zeros32 = jnp.zeros((CHUNK, 32), f32)

  hist = _sc_hist(col_h, zeros16, ones16)

  x = pl.pallas_call(
      _features_body,
      grid=(GRID,),
      in_specs=[
          pl.BlockSpec((1, 1, BLK), lambda i: (i, 0, 0)),
          pl.BlockSpec((1, L, BLK), lambda i: (i, 0, 0)),
          _full((TAG_V, ED_TAG)),
          _full((ATTR_V, ED_ATTR)),
          _full((4 * H, ED_ATTR)),
          _full((4 * H, H)),
          _full((1, 4 * H)),
      ],
      out_specs=_rows(GC),
      out_shape=jax.ShapeDtypeStruct((NP, GC), f32),
  )(tags_p, attrs_p, tag_table, attr_table, W_ih, W_hh,
    (b_ih + b_hh).reshape(1, 4 * H))

  hist_spec = pl.BlockSpec((NSC, BLK, 16), lambda i: (0, i, 0))

  y1_lo, y1_hi = pl.pallas_call(
      _prep1_body,
      grid=(GRID,),
      in_specs=[_rows(GC), hist_spec, _full((GC, GC))],
      out_specs=(_rows(32), _rows(32)),
      out_shape=(jax.ShapeDtypeStruct((NP, 32), f32),
                 jax.ShapeDtypeStruct((NP, 32), f32)),
  )(x, hist, W1)

  a1_lo, a1_hi = _sc_conv(y1_lo, y1_hi, row_c, col_c, zeros32)

  y2_lo, y2_hi = pl.pallas_call(
      _mid_body,
      grid=(GRID,),
      in_specs=[_rows(32), _rows(32), _rows(32), _rows(32), hist_spec,
                _full((1, GC)), _full((GC, GC))],
      out_specs=(_rows(32), _rows(32)),
      out_shape=(jax.ShapeDtypeStruct((NP, 32), f32),
                 jax.ShapeDtypeStruct((NP, 32), f32)),
  )(a1_lo, a1_hi, y1_lo, y1_hi, hist, b1.reshape(1, GC), W2)

  a2_lo, a2_hi = _sc_conv(y2_lo, y2_hi, row_c, col_c, zeros32)

  out = pl.pallas_call(
      _pool_body,
      grid=(GRID,),
      in_specs=[_rows(32), _rows(32), _rows(32), _rows(32), hist_spec,
                _full((1, GC)),
                pl.BlockSpec((1, 1, BLK), lambda i: (i, 0, 0)),
                _full((C, GC)), _full((1, C))],
      out_specs=pl.BlockSpec((B, C), lambda i: (0, 0)),
      out_shape=jax.ShapeDtypeStruct((B, C), f32),
      scratch_shapes=[pltpu.VMEM((B, GC + 8), f32)],
  )(a2_lo, a2_hi, y2_lo, y2_hi, hist, b2.reshape(1, GC), batch_p, Wfc,
    bfc.reshape(1, C))

  return out


# speedup vs baseline: 11.5614x; 11.5614x over previous
import functools

import jax
import jax.numpy as jnp
from jax import lax
from jax.experimental import pallas as pl
from jax.experimental.pallas import tpu as pltpu
from jax.experimental.pallas import tpu_sc as plsc

N = 50000
E = 800000
L = 8
TAG_V = 128
ATTR_V = 256
ED_TAG = 16
ED_ATTR = 32
H = 48
GC = 64
C = 16
B = 512

BLK = 2048
GRID = 25
NP = BLK * GRID
NSC = 2
NTILE = 16
CHUNK = 128
JROWS = 8
EP = 819200
ROWS_CONV = EP // NTILE // CHUNK
ROWS_HIST = EP // (NSC * NTILE) // CHUNK
STRIPE = NP // NTILE

_mesh = plsc.VectorSubcoreMesh(
    core_axis_name="c", subcore_axis_name="s", num_cores=NSC,
    num_subcores=NTILE)
_sc_params = pltpu.CompilerParams(use_tc_tiling_on_sc=False)


def _sc_hist(col_idx, zeros16, ones16):
  @functools.partial(
      pl.kernel,
      out_type=jax.ShapeDtypeStruct((NSC, NP, 16), jnp.float32),
      mesh=_mesh,
      compiler_params=_sc_params,
      scratch_types=[
          pltpu.VMEM((JROWS, CHUNK), jnp.int32),
          pltpu.VMEM((CHUNK, 16), jnp.float32),
          pltpu.VMEM((CHUNK, 16), jnp.float32),
          pltpu.VMEM_SHARED((NP, 16), jnp.float32),
      ],
  )
  def k(col_hbm, z_hbm, ones_hbm, out_hbm, cv, zb, onesb, acc):
    cid = lax.axis_index("c")
    sid = lax.axis_index("s")
    pltpu.sync_copy(z_hbm, zb)
    pltpu.sync_copy(ones_hbm, onesb)

    @pl.loop(0, STRIPE // CHUNK)
    def _(i):
      pltpu.sync_copy(zb, acc.at[pl.ds(sid * STRIPE + i * CHUNK, CHUNK)])

    plsc.subcore_barrier()

    @pl.loop(0, ROWS_HIST // JROWS)
    def _(ci):
      pltpu.sync_copy(col_hbm.at[cid, sid, pl.ds(ci * JROWS, JROWS)], cv)
      for j in range(JROWS):
        pltpu.sync_copy(onesb, acc.at[cv.at[j]], add=True)

    plsc.subcore_barrier()
    pltpu.sync_copy(acc.at[pl.ds(sid * STRIPE, STRIPE)],
                    out_hbm.at[cid, pl.ds(sid * STRIPE, STRIPE)])

  return k(col_idx, zeros16, ones16)


def _sc_conv(y_lo, y_hi, row_idx, col_idx, zeros32):
  @functools.partial(
      pl.kernel,
      out_type=(jax.ShapeDtypeStruct((NP, 32), jnp.float32),
                jax.ShapeDtypeStruct((NP, 32), jnp.float32)),
      mesh=_mesh,
      compiler_params=_sc_params,
      scratch_types=[
          pltpu.VMEM((JROWS, CHUNK), jnp.int32),
          pltpu.VMEM((JROWS, CHUNK), jnp.int32),
          pltpu.VMEM((CHUNK, 32), jnp.float32),
          pltpu.VMEM((CHUNK, 32), jnp.float32),
          pltpu.VMEM_SHARED((NP, 32), jnp.float32),
      ],
  )
  def k(ylo_hbm, yhi_hbm, row_hbm, col_hbm, z_hbm, olo_hbm, ohi_hbm,
        rv, cv, gb, zb, acc):
    cid = lax.axis_index("c")
    sid = lax.axis_index("s")
    pltpu.sync_copy(z_hbm, zb)

    @pl.loop(0, STRIPE // CHUNK)
    def _(i):
      pltpu.sync_copy(zb, acc.at[pl.ds(sid * STRIPE + i * CHUNK, CHUNK)])

    plsc.subcore_barrier()

    def edge_pass(y_hbm):
      @pl.loop(0, ROWS_CONV // JROWS)
      def _(ci):
        pltpu.sync_copy(row_hbm.at[sid, pl.ds(ci * JROWS, JROWS)], rv)
        pltpu.sync_copy(col_hbm.at[sid, pl.ds(ci * JROWS, JROWS)], cv)
        for j in range(JROWS):
          pltpu.sync_copy(y_hbm.at[rv.at[j]], gb)
          pltpu.sync_copy(gb, acc.at[cv.at[j]], add=True)

    @pl.when(cid == 0)
    def _():
      edge_pass(ylo_hbm)

    @pl.when(cid == 1)
    def _():
      edge_pass(yhi_hbm)

    plsc.subcore_barrier()

    @pl.when(cid == 0)
    def _():
      pltpu.sync_copy(acc.at[pl.ds(sid * STRIPE, STRIPE)],
                      olo_hbm.at[pl.ds(sid * STRIPE, STRIPE)])

    @pl.when(cid == 1)
    def _():
      pltpu.sync_copy(acc.at[pl.ds(sid * STRIPE, STRIPE)],
                      ohi_hbm.at[pl.ds(sid * STRIPE, STRIPE)])

  return k(y_lo, y_hi, row_idx, col_idx, zeros32)


def _dinv_from_hist(hist_blk):
  deg = hist_blk[0, :, 0:1] + hist_blk[1, :, 0:1] + 1.0
  return lax.rsqrt(deg)


def _features_body(tags_ref, attrs_ref, tag_t_ref, attr_t_ref, wih_ref,
                   whh_ref, bias_ref, x_ref):
  tags = tags_ref[0, 0, :]
  oh_t = (tags[:, None] == lax.broadcasted_iota(jnp.int32, (BLK, TAG_V), 1)
          ).astype(jnp.float32)
  tag_e = jnp.dot(oh_t, tag_t_ref[...], preferred_element_type=jnp.float32)

  wih = wih_ref[...]
  whh = whh_ref[...]
  bias = bias_ref[...]
  h = jnp.zeros((BLK, H), jnp.float32)
  c = jnp.zeros((BLK, H), jnp.float32)
  for l in range(L):
    al = attrs_ref[0, l, :]
    oh_a = (al[:, None] == lax.broadcasted_iota(jnp.int32, (BLK, ATTR_V), 1)
            ).astype(jnp.float32)
    xt = jnp.dot(oh_a, attr_t_ref[...], preferred_element_type=jnp.float32)
    g = (lax.dot_general(xt, wih, (((1,), (1,)), ((), ())),
                         preferred_element_type=jnp.float32)
         + lax.dot_general(h, whh, (((1,), (1,)), ((), ())),
                           preferred_element_type=jnp.float32)
         + bias)
    gi = jax.nn.sigmoid(g[:, 0:H])
    gf = jax.nn.sigmoid(g[:, H:2 * H])
    gg = jnp.tanh(g[:, 2 * H:3 * H])
    go = jax.nn.sigmoid(g[:, 3 * H:4 * H])
    c = gf * c + gi * gg
    h = go * jnp.tanh(c)
  x_ref[...] = jnp.concatenate([tag_e, h], axis=1)


def _prep1_body(x_ref, hist_ref, w1_ref, ylo_ref, yhi_ref):
  dinv = _dinv_from_hist(hist_ref[...])
  xw = lax.dot_general(x_ref[...], w1_ref[...], (((1,), (1,)), ((), ())),
                       preferred_element_type=jnp.float32)
  y = dinv * xw
  ylo_ref[...] = y[:, :32]
  yhi_ref[...] = y[:, 32:]


def _mid_body(alo_ref, ahi_ref, ylo_ref, yhi_ref, hist_ref, b1_ref, w2_ref,
              y2lo_ref, y2hi_ref):
  dinv = _dinv_from_hist(hist_ref[...])
  acc = jnp.concatenate([alo_ref[...], ahi_ref[...]], axis=1)
  y1 = jnp.concatenate([ylo_ref[...], yhi_ref[...]], axis=1)
  h1 = jax.nn.relu(dinv * (acc + y1) + b1_ref[...])
  xw2 = lax.dot_general(h1, w2_ref[...], (((1,), (1,)), ((), ())),
                        preferred_element_type=jnp.float32)
  y2 = dinv * xw2
  y2lo_ref[...] = y2[:, :32]
  y2hi_ref[...] = y2[:, 32:]


def _pool_body(alo_ref, ahi_ref, ylo_ref, yhi_ref, hist_ref, b2_ref,
               batch_ref, wfc_ref, bfc_ref, out_ref, acc_ref):
  i = pl.program_id(0)

  @pl.when(i == 0)
  def _():
    acc_ref[...] = jnp.zeros_like(acc_ref)

  dinv = _dinv_from_hist(hist_ref[...])
  acc = jnp.concatenate([alo_ref[...], ahi_ref[...]], axis=1)
  y2 = jnp.concatenate([ylo_ref[...], yhi_ref[...]], axis=1)
  h2 = jax.nn.relu(dinv * (acc + y2) + b2_ref[...])
  h2a = jnp.concatenate([h2, jnp.ones((BLK, 8), jnp.float32)], axis=1)
  bb = batch_ref[0, 0, :]
  oh = (bb[:, None] == lax.broadcasted_iota(jnp.int32, (BLK, B), 1)
        ).astype(jnp.float32)
  acc_ref[...] += lax.dot_general(oh, h2a, (((0,), (0,)), ((), ())),
                                  preferred_element_type=jnp.float32)

  @pl.when(i == GRID - 1)
  def _():
    sums = acc_ref[..., :GC]
    cnt = jnp.maximum(acc_ref[..., GC:GC + 1], 1.0)
    pooled = sums / cnt
    out_ref[...] = (lax.dot_general(
        pooled, wfc_ref[...], (((1,), (1,)), ((), ())),
        preferred_element_type=jnp.float32) + bfc_ref[...])


def _full(shape):
  return pl.BlockSpec(shape, lambda i: tuple(0 for _ in shape))


def _rows(width):
  return pl.BlockSpec((BLK, width), lambda i: (i, 0))


def kernel(tags, attrs, edge_index, batch, tag_table, attr_table, W_ih, W_hh,
           b_ih, b_hh, W1, b1, W2, b2, Wfc, bfc):
  f32 = jnp.float32
  tags_p = jnp.pad(tags.astype(jnp.int32), (0, NP - N)).reshape(GRID, 1, BLK)
  attrs_p = jnp.pad(attrs.astype(jnp.int32), ((0, NP - N), (0, 0)))
  attrs_p = attrs_p.T.reshape(L, GRID, BLK).transpose(1, 0, 2)
  batch_p = jnp.pad(batch.astype(jnp.int32), (0, NP - N),
                    constant_values=B).reshape(GRID, 1, BLK)
  row = jnp.pad(edge_index[0].astype(jnp.int32), (0, EP - E))
  col = jnp.pad(edge_index[1].astype(jnp.int32), (0, EP - E),
                constant_values=NP - 1)
  row_c = row.reshape(NTILE, ROWS_CONV, CHUNK)
  col_c = col.reshape(NTILE, ROWS_CONV, CHUNK)
  col_h = col.reshape(NSC, NTILE, ROWS_HIST, CHUNK)
  ones16 = jnp.ones((CHUNK, 16), f32)
  zeros16 = jnp.zeros((CHUNK, 16), f32)
  zeros32 = jnp.zeros((CHUNK, 32), f32)

  hist = _sc_hist(col_h, zeros16, ones16)

  x = pl.pallas_call(
      _features_body,
      grid=(GRID,),
      in_specs=[
          pl.BlockSpec((1, 1, BLK), lambda i: (i, 0, 0)),
          pl.BlockSpec((1, L, BLK), lambda i: (i, 0, 0)),
          _full((TAG_V, ED_TAG)),
          _full((ATTR_V, ED_ATTR)),
          _full((4 * H, ED_ATTR)),
          _full((4 * H, H)),
          _full((1, 4 * H)),
      ],
      out_specs=_rows(GC),
      out_shape=jax.ShapeDtypeStruct((NP, GC), f32),
  )(tags_p, attrs_p, tag_table, attr_table, W_ih, W_hh,
    (b_ih + b_hh).reshape(1, 4 * H))

  hist_spec = pl.BlockSpec((NSC, BLK, 16), lambda i: (0, i, 0))

  y1_lo, y1_hi = pl.pallas_call(
      _prep1_body,
      grid=(GRID,),
      in_specs=[_rows(GC), hist_spec, _full((GC, GC))],
      out_specs=(_rows(32), _rows(32)),
      out_shape=(jax.ShapeDtypeStruct((NP, 32), f32),
                 jax.ShapeDtypeStruct((NP, 32), f32)),
  )(x, hist, W1)

  a1_lo, a1_hi = _sc_conv(y1_lo, y1_hi, row_c, col_c, zeros32)

  y2_lo, y2_hi = pl.pallas_call(
      _mid_body,
      grid=(GRID,),
      in_specs=[_rows(32), _rows(32), _rows(32), _rows(32), hist_spec,
                _full((1, GC)), _full((GC, GC))],
      out_specs=(_rows(32), _rows(32)),
      out_shape=(jax.ShapeDtypeStruct((NP, 32), f32),
                 jax.ShapeDtypeStruct((NP, 32), f32)),
  )(a1_lo, a1_hi, y1_lo, y1_hi, hist, b1.reshape(1, GC), W2)

  a2_lo, a2_hi = _sc_conv(y2_lo, y2_hi, row_c, col_c, zeros32)

  out = pl.pallas_call(
      _pool_body,
      grid=(GRID,),
      in_specs=[_rows(32), _rows(32), _rows(32), _rows(32), hist_spec,
                _full((1, GC)),
                pl.BlockSpec((1, 1, BLK), lambda i: (i, 0, 0)),
                _full((C, GC)), _full((1, C))],
      out_specs=pl.BlockSpec((B, C), lambda i: (0, 0)),
      out_shape=jax.ShapeDtypeStruct((B, C), f32),
      scratch_shapes=[pltpu.VMEM((B, GC + 8), f32)],
  )(a2_lo, a2_hi, y2_lo, y2_hi, hist, b2.reshape(1, GC), batch_p, Wfc,
    bfc.reshape(1, C))

  return out


# speedup vs baseline: 16.3513x; 1.4143x over previous
import functools

import jax
import jax.numpy as jnp
from jax import lax
from jax.experimental import pallas as pl
from jax.experimental.pallas import tpu as pltpu
from jax.experimental.pallas import tpu_sc as plsc

N = 50000
E = 800000
L = 8
TAG_V = 128
ATTR_V = 256
ED_TAG = 16
ED_ATTR = 32
H = 48
GC = 64
C = 16
B = 512

BLK = 2048
GRID = 25
NP = BLK * GRID
NSC = 2
NTILE = 16
CHUNK = 128
JROWS = 8
EP = 819200
ROWS_HIST = EP // (NSC * NTILE) // CHUNK
STRIPE = NP // NTILE
SUPER = EP // NTILE // CHUNK // JROWS

_mesh = plsc.VectorSubcoreMesh(
    core_axis_name="c", subcore_axis_name="s", num_cores=NSC,
    num_subcores=NTILE)
_sc_params = pltpu.CompilerParams(use_tc_tiling_on_sc=False)


def _sc_hist(col_idx, zeros16, ones16):
  @functools.partial(
      pl.kernel,
      out_type=jax.ShapeDtypeStruct((NSC, NP, 16), jnp.float32),
      mesh=_mesh,
      compiler_params=_sc_params,
      scratch_types=[
          pltpu.VMEM((JROWS, CHUNK), jnp.int32),
          pltpu.VMEM((CHUNK, 16), jnp.float32),
          pltpu.VMEM((CHUNK, 16), jnp.float32),
          pltpu.VMEM_SHARED((NP, 16), jnp.float32),
      ],
  )
  def k(col_hbm, z_hbm, ones_hbm, out_hbm, cv, zb, onesb, acc):
    cid = lax.axis_index("c")
    sid = lax.axis_index("s")
    pltpu.sync_copy(z_hbm, zb)
    pltpu.sync_copy(ones_hbm, onesb)

    @pl.loop(0, STRIPE // CHUNK)
    def _(i):
      pltpu.sync_copy(zb, acc.at[pl.ds(sid * STRIPE + i * CHUNK, CHUNK)])

    plsc.subcore_barrier()

    @pl.loop(0, ROWS_HIST // JROWS)
    def _(ci):
      pltpu.sync_copy(col_hbm.at[cid, sid, pl.ds(ci * JROWS, JROWS)], cv)
      for j in range(JROWS):
        pltpu.sync_copy(onesb, acc.at[cv.at[j]], add=True)

    plsc.subcore_barrier()
    pltpu.sync_copy(acc.at[pl.ds(sid * STRIPE, STRIPE)],
                    out_hbm.at[cid, pl.ds(sid * STRIPE, STRIPE)])

  return k(col_idx, zeros16, ones16)


def _sc_conv(y_sc, idx_all, zeros32):
  @functools.partial(
      pl.kernel,
      out_type=(jax.ShapeDtypeStruct((NP, 32), jnp.float32),
                jax.ShapeDtypeStruct((NP, 32), jnp.float32)),
      mesh=_mesh,
      compiler_params=_sc_params,
      scratch_types=(
          [pltpu.VMEM((24, CHUNK), jnp.int32),
           pltpu.VMEM((24, CHUNK), jnp.int32)]
          + [pltpu.VMEM((CHUNK, 32), jnp.float32) for _ in range(4)]
          + [pltpu.VMEM_SHARED((NP, 32), jnp.float32)]
          + [pltpu.SemaphoreType.DMA for _ in range(10)]
      ),
  )
  def k(y_hbm, idx_hbm, z_hbm, olo_hbm, ohi_hbm, rc0, rc1, *rest):
    gb = rest[:4]
    acc = rest[4]
    si0, si1 = rest[5], rest[6]
    sg = rest[7:11]
    ss = rest[11:15]
    cid = lax.axis_index("c")
    sid = lax.axis_index("s")
    pltpu.sync_copy(z_hbm, gb[0])

    @pl.loop(0, STRIPE // CHUNK)
    def _(i):
      pltpu.sync_copy(gb[0], acc.at[pl.ds(sid * STRIPE + i * CHUNK, CHUNK)])

    plsc.subcore_barrier()

    def edge_pass(ro):
      pltpu.sync_copy(idx_hbm.at[sid, 0], rc0)
      for t in range(2):
        pltpu.async_copy(y_hbm.at[rc0.at[ro + t]], gb[t], sg[t])

      @pl.loop(0, SUPER // 2)
      def _(oc):
        for b in (0, 1):
          ci = 2 * oc + b
          rc, rcn = (rc0, rc1) if b == 0 else (rc1, rc0)
          sin_ = si1 if b == 0 else si0

          @pl.when(ci + 1 < SUPER)
          def _():
            pltpu.async_copy(idx_hbm.at[sid, ci + 1], rcn, sin_)

          for j in range(JROWS):
            jb = j % 4
            jp = (j + 2) % 4
            if j >= 6:
              pltpu.make_async_copy(gb[jp], acc.at[rc.at[16]], ss[jp]
                                    ).wait()
              @pl.when(ci + 1 < SUPER)
              def _():
                if j == 6:
                  pltpu.make_async_copy(idx_hbm.at[sid, ci + 1], rcn,
                                        sin_).wait()
                pltpu.async_copy(y_hbm.at[rcn.at[ro + j - 6]], gb[jp],
                                 sg[jp])
            elif j >= 2:
              pltpu.make_async_copy(gb[jp], acc.at[rc.at[16]], ss[jp]
                                    ).wait()
              pltpu.async_copy(y_hbm.at[rc.at[ro + j + 2]], gb[jp], sg[jp])
            else:
              @pl.when(ci > 0)
              def _():
                pltpu.make_async_copy(gb[jp], acc.at[rc.at[16]], ss[jp]
                                      ).wait()
              pltpu.async_copy(y_hbm.at[rc.at[ro + j + 2]], gb[jp], sg[jp])
            pltpu.make_async_copy(y_hbm.at[rc.at[ro + j]], gb[jb],
                                  sg[jb]).wait()
            pltpu.async_copy(gb[jb], acc.at[rc.at[16 + j]], ss[jb],
                             add=True)

      for t in (2, 3):
        pltpu.make_async_copy(gb[t], acc.at[rc1.at[16]], ss[t]).wait()

    @pl.when(cid == 0)
    def _():
      edge_pass(0)

    @pl.when(cid == 1)
    def _():
      edge_pass(JROWS)

    plsc.subcore_barrier()

    @pl.when(cid == 0)
    def _():
      pltpu.sync_copy(acc.at[pl.ds(sid * STRIPE, STRIPE)],
                      olo_hbm.at[pl.ds(sid * STRIPE, STRIPE)])

    @pl.when(cid == 1)
    def _():
      pltpu.sync_copy(acc.at[pl.ds(sid * STRIPE, STRIPE)],
                      ohi_hbm.at[pl.ds(sid * STRIPE, STRIPE)])

  return k(y_sc, idx_all, zeros32)


def _dinv_from_hist(hr):
  s = hr[0] + hr[1]
  deg = jnp.concatenate([s[:, 16 * k:16 * k + 1] for k in range(8)],
                        axis=0) + 1.0
  return lax.rsqrt(deg)


def _features_body(tags_ref, attrs_ref, tag_t_ref, attr_t_ref, wih_ref,
                   whh_ref, bias_ref, hist_ref, w1_ref, yp_ref):
  bf16 = jnp.bfloat16
  tags = tags_ref[0, 0, :]
  oh_t = (tags[:, None] == lax.broadcasted_iota(jnp.int32, (BLK, TAG_V), 1)
          ).astype(jnp.float32).astype(bf16)
  tag_e = jnp.dot(oh_t, tag_t_ref[...].astype(bf16),
                  preferred_element_type=jnp.float32)

  aw = lax.dot_general(attr_t_ref[...], wih_ref[...], (((1,), (1,)), ((), ())),
                       preferred_element_type=jnp.float32).astype(bf16)
  whh = whh_ref[...].astype(bf16)
  bias = bias_ref[...]
  li = lax.broadcasted_iota(jnp.int32, (1, 4 * H), 1)
  is_g = jnp.logical_and(li >= 2 * H, li < 3 * H)
  pre = jnp.where(is_g, 1.0, 0.5).astype(jnp.float32)
  post_m = jnp.where(is_g, 1.0, 0.5).astype(jnp.float32)
  post_a = jnp.where(is_g, 0.0, 0.5).astype(jnp.float32)

  h = jnp.zeros((BLK, H), jnp.float32)
  c = jnp.zeros((BLK, H), jnp.float32)
  for l in range(L):
    al = attrs_ref[0, l, :]
    oh_a = (al[:, None] == lax.broadcasted_iota(jnp.int32, (BLK, ATTR_V), 1)
            ).astype(jnp.float32).astype(bf16)
    g = (jnp.dot(oh_a, aw, preferred_element_type=jnp.float32)
         + lax.dot_general(h.astype(bf16), whh, (((1,), (1,)), ((), ())),
                           preferred_element_type=jnp.float32)
         + bias)
    act = jnp.tanh(g * pre) * post_m + post_a
    gi = act[:, 0:H]
    gf = act[:, H:2 * H]
    gg = act[:, 2 * H:3 * H]
    go = act[:, 3 * H:4 * H]
    c = gf * c + gi * gg
    h = go * jnp.tanh(c)

  x = jnp.concatenate([tag_e, h], axis=1)
  dinv = _dinv_from_hist(hist_ref[...])
  xw = lax.dot_general(x, w1_ref[...], (((1,), (1,)), ((), ())),
                       preferred_element_type=jnp.float32)
  y = dinv * xw
  yp_ref[...] = jnp.concatenate([y[:BLK // 2, :], y[BLK // 2:, :]], axis=1)


def _unpack_acc(ref):
  b = ref[...]
  return jnp.concatenate([b[:, 32 * k:32 * (k + 1)] for k in range(4)],
                         axis=0)


def _unpack_y(ref):
  b = ref[...]
  return jnp.concatenate([b[:, :GC], b[:, GC:]], axis=0)


def _mid_body(alo_ref, ahi_ref, yp_ref, hist_ref, b1_ref, w2_ref, y2p_ref):
  dinv = _dinv_from_hist(hist_ref[...])
  acc = jnp.concatenate([_unpack_acc(alo_ref), _unpack_acc(ahi_ref)], axis=1)
  y1 = _unpack_y(yp_ref)
  h1 = jax.nn.relu(dinv * (acc + y1) + b1_ref[...])
  xw2 = lax.dot_general(h1, w2_ref[...], (((1,), (1,)), ((), ())),
                        preferred_element_type=jnp.float32)
  y2 = dinv * xw2
  y2p_ref[...] = jnp.concatenate([y2[:BLK // 2, :], y2[BLK // 2:, :]],
                                 axis=1)


def _pool_body(alo_ref, ahi_ref, yp_ref, hist_ref, b2_ref,
               batch_ref, wfc_ref, bfc_ref, out_ref, acc_ref):
  i = pl.program_id(0)

  @pl.when(i == 0)
  def _():
    acc_ref[...] = jnp.zeros_like(acc_ref)

  dinv = _dinv_from_hist(hist_ref[...])
  acc = jnp.concatenate([_unpack_acc(alo_ref), _unpack_acc(ahi_ref)], axis=1)
  y2 = _unpack_y(yp_ref)
  h2 = jax.nn.relu(dinv * (acc + y2) + b2_ref[...])
  h2a = jnp.concatenate([h2, jnp.ones((BLK, 8), jnp.float32)], axis=1)
  bb = batch_ref[0, 0, :]
  oh = (bb[:, None] == lax.broadcasted_iota(jnp.int32, (BLK, B), 1)
        ).astype(jnp.float32)
  acc_ref[...] += lax.dot_general(oh, h2a, (((0,), (0,)), ((), ())),
                                  preferred_element_type=jnp.float32)

  @pl.when(i == GRID - 1)
  def _():
    sums = acc_ref[..., :GC]
    cnt = jnp.maximum(acc_ref[..., GC:GC + 1], 1.0)
    pooled = sums / cnt
    out_ref[...] = (lax.dot_general(
        pooled, wfc_ref[...], (((1,), (1,)), ((), ())),
        preferred_element_type=jnp.float32) + bfc_ref[...])


def _full(shape):
  return pl.BlockSpec(shape, lambda i: tuple(0 for _ in shape))


_hist_spec = pl.BlockSpec((NSC, BLK // 8, 128), lambda i: (0, i, 0))
_yp_spec = pl.BlockSpec((BLK // 2, 128), lambda i: (i, 0))
_acc_spec = pl.BlockSpec((BLK // 4, 128), lambda i: (i, 0))


def kernel(tags, attrs, edge_index, batch, tag_table, attr_table, W_ih, W_hh,
           b_ih, b_hh, W1, b1, W2, b2, Wfc, bfc):
  f32 = jnp.float32
  tags_p = jnp.pad(tags.astype(jnp.int32), (0, NP - N)).reshape(GRID, 1, BLK)
  attrs_p = jnp.pad(attrs.astype(jnp.int32), ((0, NP - N), (0, 0)))
  attrs_p = attrs_p.T.reshape(L, GRID, BLK).transpose(1, 0, 2)
  batch_p = jnp.pad(batch.astype(jnp.int32), (0, NP - N),
                    constant_values=B).reshape(GRID, 1, BLK)
  row = jnp.pad(edge_index[0].astype(jnp.int32), (0, EP - E))
  col = jnp.pad(edge_index[1].astype(jnp.int32), (0, EP - E),
                constant_values=NP - 1)
  ri, rloc = row >> 11, row & 2047
  ybase = ri * 4096 + (rloc & 1023) * 4 + ((rloc >> 10) << 1)
  ci, cloc = col >> 11, col & 2047
  colp = (ci * 512 + (cloc & 511)) * 4 + (cloc >> 9)
  colh = (ci * 256 + (cloc & 255)) * 8 + (cloc >> 8)
  idx_all = jnp.concatenate(
      [ybase.reshape(NTILE, SUPER, JROWS, CHUNK),
       (ybase + 1).reshape(NTILE, SUPER, JROWS, CHUNK),
       colp.reshape(NTILE, SUPER, JROWS, CHUNK)], axis=2)
  col_h = colh.reshape(NSC, NTILE, ROWS_HIST, CHUNK)
  ones16 = jnp.ones((CHUNK, 16), f32)
  zeros16 = jnp.zeros((CHUNK, 16), f32)
  zeros32 = jnp.zeros((CHUNK, 32), f32)

  hist = _sc_hist(col_h, zeros16, ones16).reshape(NSC, NP // 8, 128)

  y1p = pl.pallas_call(
      _features_body,
      grid=(GRID,),
      in_specs=[
          pl.BlockSpec((1, 1, BLK), lambda i: (i, 0, 0)),
          pl.BlockSpec((1, L, BLK), lambda i: (i, 0, 0)),
          _full((TAG_V, ED_TAG)),
          _full((ATTR_V, ED_ATTR)),
          _full((4 * H, ED_ATTR)),
          _full((4 * H, H)),
          _full((1, 4 * H)),
          _hist_spec,
          _full((GC, GC)),
      ],
      out_specs=_yp_spec,
      out_shape=jax.ShapeDtypeStruct((NP // 2, 128), f32),
  )(tags_p, attrs_p, tag_table, attr_table, W_ih, W_hh,
    (b_ih + b_hh).reshape(1, 4 * H), hist, W1)

  a1_lo, a1_hi = _sc_conv(y1p.reshape(2 * NP, 32), idx_all, zeros32)

  y2p = pl.pallas_call(
      _mid_body,
      grid=(GRID,),
      in_specs=[_acc_spec, _acc_spec, _yp_spec, _hist_spec,
                _full((1, GC)), _full((GC, GC))],
      out_specs=_yp_spec,
      out_shape=jax.ShapeDtypeStruct((NP // 2, 128), f32),
  )(a1_lo.reshape(NP // 4, 128), a1_hi.reshape(NP // 4, 128), y1p, hist,
    b1.reshape(1, GC), W2)

  a2_lo, a2_hi = _sc_conv(y2p.reshape(2 * NP, 32), idx_all, zeros32)

  out = pl.pallas_call(
      _pool_body,
      grid=(GRID,),
      in_specs=[_acc_spec, _acc_spec, _yp_spec, _hist_spec,
                _full((1, GC)),
                pl.BlockSpec((1, 1, BLK), lambda i: (i, 0, 0)),
                _full((C, GC)), _full((1, C))],
      out_specs=pl.BlockSpec((B, C), lambda i: (0, 0)),
      out_shape=jax.ShapeDtypeStruct((B, C), f32),
      scratch_shapes=[pltpu.VMEM((B, GC + 8), f32)],
  )(a2_lo.reshape(NP // 4, 128), a2_hi.reshape(NP // 4, 128), y2p, hist,
    b2.reshape(1, GC), batch_p, Wfc, bfc.reshape(1, C))

  return out
